# Initial kernel scaffold; baseline (speedup 1.0000x reference)
import jax
import jax.numpy as jnp
from jax.experimental import pallas as pl


def kernel(pred_cls_logit, pred_delta, img_width):
    raise NotImplementedError("write your pallas kernel here")


# speedup vs baseline: 1.3892x; 1.3892x over previous
import functools

import jax
import jax.numpy as jnp
import numpy as np
from jax import lax
from jax.experimental import pallas as pl
from jax.experimental.pallas import tpu as pltpu
from jax.experimental.pallas import tpu_sc as plsc

B = 8
FW = 20000
FWP = 20096
NBLK = 157
NBLK_P = 160
BLK = 128
MAX_OUT = 50
NEG = np.float32(-1e30)
NEG_HALF = np.float32(-5e29)
DIST = np.float32(16.0)
THR = np.float32(0.7)
LOG2E = np.float32(1.4426950408889634)
SEL_PAD = np.float32(3e38)


def _prep_body(logit_ref, d0_ref, d1_ref, wl_ref, m_ref, cen_ref):
    x = logit_ref[...]
    e = jnp.exp2(x * (-LOG2E))
    s = 1.0 / (1.0 + e)
    iot = lax.broadcasted_iota(jnp.int32, (B, FWP), 1).astype(jnp.float32)
    ic = (iot + 0.5) * 16.0
    p0 = d0_ref[...] * 16.0 + ic
    p1 = d1_ref[...] * 16.0 + ic
    wl = wl_ref[0]
    valid = (p0 >= 0.0) & (p0 <= wl) & (p1 >= 0.0) & (p1 <= wl) & (s >= THR)
    m_ref[...] = jnp.where(valid, s, NEG)
    cen_ref[...] = (p0 + p1) * 0.5


def _st1(ref, idx, val):
    iota = lax.broadcasted_iota(jnp.int32, (16,), 0)
    idxv = jnp.zeros((16,), jnp.int32) + idx
    valv = jnp.zeros((16,), jnp.float32) + val
    plsc.store_scatter(ref, [idxv], valv, mask=iota == 0)


def _nms_walk_body(m_hbm, cen_hbm, dfl_hbm, pos_hbm, sc_hbm,
                   m_loc, cen_loc, dfl_loc, bm_loc, selc_loc,
                   pos_loc, sc_loc):
    wid = lax.axis_index("s") * 2 + lax.axis_index("c")

    @pl.when(wid < B)
    def _run():
        b = wid
        zero16 = jnp.zeros((16,), jnp.float32)
        for i in range(16):
            pos_loc[pl.ds(i * 16, 16)] = zero16
        for i in range(8):
            sc_loc[pl.ds(i * 16, 16)] = zero16
        for i in range(4):
            selc_loc[pl.ds(i * 16, 16)] = jnp.full((16,), SEL_PAD, jnp.float32)

        pltpu.sync_copy(m_hbm.at[pl.ds(pl.multiple_of(b * FWP, 8), FWP)],
                        m_loc)
        pltpu.sync_copy(cen_hbm.at[pl.ds(pl.multiple_of(b * FWP, 8), FWP)],
                        cen_loc)
        pltpu.sync_copy(dfl_hbm.at[pl.ds(pl.multiple_of(b * 2 * FW, 8),
                                         2 * FW)],
                        dfl_loc.at[pl.ds(0, 2 * FW)])

        bm_loc[pl.ds(144, 16)] = jnp.full((16,), NEG, jnp.float32)

        def _bm(k, _):
            base = k * BLK
            acc = m_loc[pl.ds(base, 16)]
            for j in range(1, 8):
                acc = jnp.maximum(acc, m_loc[pl.ds(base + j * 16, 16)])
            _st1(bm_loc, k, jnp.max(acc))
            return 0

        lax.fori_loop(0, NBLK, _bm, 0)

        iota16 = lax.broadcasted_iota(jnp.int32, (16,), 0)
        big = jnp.int32(10**9)

        def _cond(carry):
            cnt, done = carry
            return (cnt < MAX_OUT) & jnp.logical_not(done)

        def _step(carry):
            cnt, _ = carry
            gm = bm_loc[pl.ds(0, 16)]
            for j in range(1, 10):
                gm = jnp.maximum(gm, bm_loc[pl.ds(j * 16, 16)])
            mval = jnp.max(gm)
            exhausted = mval <= NEG_HALF
            bidx = jnp.full((16,), big, jnp.int32)
            for j in range(10):
                v = bm_loc[pl.ds(j * 16, 16)]
                bidx = jnp.minimum(
                    bidx, jnp.where(v == mval, j * 16 + iota16, big))
            b_star = jnp.min(bidx)
            base = b_star * BLK
            qv = jnp.full((16,), big, jnp.int32)
            for j in range(8):
                off = base + j * 16
                v = m_loc[pl.ds(off, 16)]
                qv = jnp.minimum(qv, jnp.where(v == mval, off + iota16, big))
            q = jnp.min(qv)
            zeros16 = jnp.zeros((16,), jnp.int32)
            cvec = plsc.load_gather(cen_loc, [q + zeros16])
            dm = jnp.abs(selc_loc[pl.ds(0, 16)] - cvec)
            for j in range(1, 4):
                dm = jnp.minimum(
                    dm, jnp.abs(selc_loc[pl.ds(j * 16, 16)] - cvec))
            keep = (jnp.min(dm) > DIST) & jnp.logical_not(exhausted)

            @pl.when(jnp.logical_not(exhausted))
            def _pop():
                _st1(m_loc, q, NEG)
                acc = m_loc[pl.ds(base, 16)]
                for j in range(1, 8):
                    acc = jnp.maximum(acc, m_loc[pl.ds(base + j * 16, 16)])
                _st1(bm_loc, b_star, jnp.max(acc))

            @pl.when(keep)
            def _emit():
                iotav = jnp.zeros((16,), jnp.int32) + cnt
                plsc.store_scatter(selc_loc, [iotav], cvec, mask=iota16 == 0)
                d0v = plsc.load_gather(dfl_loc, [2 * q + zeros16])
                d1v = plsc.load_gather(dfl_loc, [2 * q + 1 + zeros16])
                icq = (q.astype(jnp.float32) + 0.5) * 16.0
                p0 = d0v * 16.0 + icq
                p1 = d1v * 16.0 + icq
                posv = jnp.where(iota16 == 0, p0,
                                 jnp.where(iota16 == 1, p1, 1.0))
                plsc.store_scatter(pos_loc, [4 * cnt + iota16], posv,
                                   mask=iota16 < 3)
                scv = jnp.where(iota16 == 0, mval, 1.0)
                plsc.store_scatter(sc_loc, [2 * cnt + iota16], scv,
                                   mask=iota16 < 2)

            cnt_next = cnt + jnp.where(keep, 1, 0).astype(jnp.int32)
            return cnt_next, exhausted

        lax.while_loop(_cond, _step, (jnp.int32(0), jnp.bool_(False)))

        pltpu.sync_copy(pos_loc,
                        pos_hbm.at[pl.ds(pl.multiple_of(b * 256, 8), 256)])
        pltpu.sync_copy(sc_loc,
                        sc_hbm.at[pl.ds(pl.multiple_of(b * 128, 8), 128)])


def kernel(pred_cls_logit, pred_delta, img_width):
    wl = (jnp.float32(img_width) - 1.0).reshape(1)
    d0 = pred_delta[:, :, 0]
    d1 = pred_delta[:, :, 1]
    pad = ((0, 0), (0, FWP - FW))
    logit_p = jnp.pad(pred_cls_logit, pad, constant_values=-100.0)
    d0p = jnp.pad(d0, pad)
    d1p = jnp.pad(d1, pad)

    m, cen = pl.pallas_call(
        _prep_body,
        out_shape=[
            jax.ShapeDtypeStruct((B, FWP), jnp.float32),
            jax.ShapeDtypeStruct((B, FWP), jnp.float32),
        ],
        in_specs=[
            pl.BlockSpec(memory_space=pltpu.VMEM),
            pl.BlockSpec(memory_space=pltpu.VMEM),
            pl.BlockSpec(memory_space=pltpu.VMEM),
            pl.BlockSpec(memory_space=pltpu.SMEM),
        ],
        out_specs=[
            pl.BlockSpec(memory_space=pltpu.VMEM),
            pl.BlockSpec(memory_space=pltpu.VMEM),
        ],
    )(logit_p, d0p, d1p, wl)

    m1 = m.reshape(B * FWP)
    cen1 = cen.reshape(B * FWP)
    dfl1 = pred_delta.reshape(B * 2 * FW)

    mesh = plsc.VectorSubcoreMesh(core_axis_name="c", subcore_axis_name="s",
                                  num_cores=2)
    pos_flat, sc_flat = pl.kernel(
        _nms_walk_body,
        out_type=[
            jax.ShapeDtypeStruct((B * 256,), jnp.float32),
            jax.ShapeDtypeStruct((B * 128,), jnp.float32),
        ],
        mesh=mesh,
        compiler_params=pltpu.CompilerParams(needs_layout_passes=False),
        scratch_types=[
            pltpu.VMEM((FWP,), jnp.float32),
            pltpu.VMEM((FWP,), jnp.float32),
            pltpu.VMEM((2 * FW + 192,), jnp.float32),
            pltpu.VMEM((NBLK_P,), jnp.float32),
            pltpu.VMEM((64,), jnp.float32),
            pltpu.VMEM((256,), jnp.float32),
            pltpu.VMEM((128,), jnp.float32),
        ],
    )(m1, cen1, dfl1)

    nms_positions = pos_flat.reshape(B, 64, 4)[:, :MAX_OUT, :3]
    nms_scores = sc_flat.reshape(B, 64, 2)[:, :MAX_OUT, :]
    return nms_positions, nms_scores


# speedup vs baseline: 4.2739x; 3.0765x over previous
import functools

import jax
import jax.numpy as jnp
import numpy as np
from jax import lax
from jax.experimental import pallas as pl
from jax.experimental.pallas import tpu as pltpu
from jax.experimental.pallas import tpu_sc as plsc

B = 8
FW = 20000
FWP = 20096
NBLK = 157
NBLK_P = 160
BLK = 128
MAX_OUT = 50
NEG = np.float32(-1e30)
NEG_HALF = np.float32(-5e29)
DIST = np.float32(16.0)
THR = np.float32(0.7)
LOG2E = np.float32(1.4426950408889634)
SEL_PAD = np.float32(3e38)


def _prep_body(logit_ref, d0_ref, d1_ref, wl_ref, m_ref, p0_ref, p1_ref):
    x = logit_ref[...]
    e = jnp.exp2(x * (-LOG2E))
    s = 1.0 / (1.0 + e)
    iot = lax.broadcasted_iota(jnp.int32, (B, FWP), 1).astype(jnp.float32)
    ic = (iot + 0.5) * 16.0
    p0 = d0_ref[...] * 16.0 + ic
    p1 = d1_ref[...] * 16.0 + ic
    wl = wl_ref[0]
    valid = (p0 >= 0.0) & (p0 <= wl) & (p1 >= 0.0) & (p1 <= wl) & (s >= THR)
    m_ref[...] = jnp.where(valid, s, NEG)
    p0_ref[...] = p0
    p1_ref[...] = p1


def _st1(ref, idx, val):
    iota = lax.broadcasted_iota(jnp.int32, (16,), 0)
    idxv = jnp.zeros((16,), jnp.int32) + idx
    valv = jnp.zeros((16,), jnp.float32) + val
    plsc.store_scatter(ref, [idxv], valv, mask=iota == 0)


def _nms_walk_body(m_hbm, p0_hbm, p1_hbm, pos_hbm, sc_hbm,
                   m_loc, p0_loc, p1_loc, bm_loc, selc_loc,
                   pos_loc, sc_loc):
    wid = lax.axis_index("s") * 2 + lax.axis_index("c")

    @pl.when(wid < B)
    def _run():
        b = wid
        zero16 = jnp.zeros((16,), jnp.float32)
        for i in range(16):
            pos_loc[pl.ds(i * 16, 16)] = zero16
        for i in range(8):
            sc_loc[pl.ds(i * 16, 16)] = zero16
        for i in range(4):
            selc_loc[pl.ds(i * 16, 16)] = jnp.full((16,), SEL_PAD, jnp.float32)

        row = pl.ds(pl.multiple_of(b * FWP, 8), FWP)
        pltpu.sync_copy(m_hbm.at[row], m_loc)
        pltpu.sync_copy(p0_hbm.at[row], p0_loc)
        pltpu.sync_copy(p1_hbm.at[row], p1_loc)

        bm_loc[pl.ds(144, 16)] = jnp.full((16,), NEG, jnp.float32)

        def _bm(k, _):
            base = k * BLK
            acc = m_loc[pl.ds(base, 16)]
            for j in range(1, 8):
                acc = jnp.maximum(acc, m_loc[pl.ds(base + j * 16, 16)])
            _st1(bm_loc, k, jnp.max(acc))
            return 0

        lax.fori_loop(0, NBLK, _bm, 0)

        iota16 = lax.broadcasted_iota(jnp.int32, (16,), 0)
        big = jnp.int32(10**9)

        def _cond(carry):
            cnt, done = carry
            return (cnt < MAX_OUT) & jnp.logical_not(done)

        def _step(carry):
            cnt, _ = carry
            gm = bm_loc[pl.ds(0, 16)]
            for j in range(1, 10):
                gm = jnp.maximum(gm, bm_loc[pl.ds(j * 16, 16)])
            mval = jnp.max(gm)
            exhausted = mval <= NEG_HALF
            bidx = jnp.full((16,), big, jnp.int32)
            for j in range(10):
                v = bm_loc[pl.ds(j * 16, 16)]
                bidx = jnp.minimum(
                    bidx, jnp.where(v == mval, j * 16 + iota16, big))
            b_star = jnp.min(bidx)
            base = b_star * BLK
            qv = jnp.full((16,), big, jnp.int32)
            for j in range(8):
                off = base + j * 16
                v = m_loc[pl.ds(off, 16)]
                qv = jnp.minimum(qv, jnp.where(v == mval, off + iota16, big))
            q = jnp.min(qv)
            zeros16 = jnp.zeros((16,), jnp.int32)
            p0v = plsc.load_gather(p0_loc, [q + zeros16])
            p1v = plsc.load_gather(p1_loc, [q + zeros16])
            cvec = (p0v + p1v) * 0.5
            dm = jnp.abs(selc_loc[pl.ds(0, 16)] - cvec)
            for j in range(1, 4):
                dm = jnp.minimum(
                    dm, jnp.abs(selc_loc[pl.ds(j * 16, 16)] - cvec))
            keep = (jnp.min(dm) > DIST) & jnp.logical_not(exhausted)

            @pl.when(jnp.logical_not(exhausted))
            def _pop():
                _st1(m_loc, q, NEG)
                acc = m_loc[pl.ds(base, 16)]
                for j in range(1, 8):
                    acc = jnp.maximum(acc, m_loc[pl.ds(base + j * 16, 16)])
                _st1(bm_loc, b_star, jnp.max(acc))

            @pl.when(keep)
            def _emit():
                iotav = jnp.zeros((16,), jnp.int32) + cnt
                plsc.store_scatter(selc_loc, [iotav], cvec, mask=iota16 == 0)
                posv = jnp.where(iota16 == 0, p0v,
                                 jnp.where(iota16 == 1, p1v, 1.0))
                plsc.store_scatter(pos_loc, [4 * cnt + iota16], posv,
                                   mask=iota16 < 3)
                scv = jnp.where(iota16 == 0, mval, 1.0)
                plsc.store_scatter(sc_loc, [2 * cnt + iota16], scv,
                                   mask=iota16 < 2)

            cnt_next = cnt + jnp.where(keep, 1, 0).astype(jnp.int32)
            return cnt_next, exhausted

        lax.while_loop(_cond, _step, (jnp.int32(0), jnp.bool_(False)))

        pltpu.sync_copy(pos_loc,
                        pos_hbm.at[pl.ds(pl.multiple_of(b * 256, 8), 256)])
        pltpu.sync_copy(sc_loc,
                        sc_hbm.at[pl.ds(pl.multiple_of(b * 128, 8), 128)])


def kernel(pred_cls_logit, pred_delta, img_width):
    wl = (jnp.float32(img_width) - 1.0).reshape(1)
    d0 = pred_delta[:, :, 0]
    d1 = pred_delta[:, :, 1]
    pad = ((0, 0), (0, FWP - FW))
    logit_p = jnp.pad(pred_cls_logit, pad, constant_values=-100.0)
    d0p = jnp.pad(d0, pad)
    d1p = jnp.pad(d1, pad)

    m, p0a, p1a = pl.pallas_call(
        _prep_body,
        out_shape=[
            jax.ShapeDtypeStruct((B, FWP), jnp.float32),
            jax.ShapeDtypeStruct((B, FWP), jnp.float32),
            jax.ShapeDtypeStruct((B, FWP), jnp.float32),
        ],
        in_specs=[
            pl.BlockSpec(memory_space=pltpu.VMEM),
            pl.BlockSpec(memory_space=pltpu.VMEM),
            pl.BlockSpec(memory_space=pltpu.VMEM),
            pl.BlockSpec(memory_space=pltpu.SMEM),
        ],
        out_specs=[
            pl.BlockSpec(memory_space=pltpu.VMEM),
            pl.BlockSpec(memory_space=pltpu.VMEM),
            pl.BlockSpec(memory_space=pltpu.VMEM),
        ],
    )(logit_p, d0p, d1p, wl)

    m1 = m.reshape(B * FWP)
    p01 = p0a.reshape(B * FWP)
    p11 = p1a.reshape(B * FWP)

    mesh = plsc.VectorSubcoreMesh(core_axis_name="c", subcore_axis_name="s",
                                  num_cores=2)
    pos_flat, sc_flat = pl.kernel(
        _nms_walk_body,
        out_type=[
            jax.ShapeDtypeStruct((B * 256,), jnp.float32),
            jax.ShapeDtypeStruct((B * 128,), jnp.float32),
        ],
        mesh=mesh,
        compiler_params=pltpu.CompilerParams(needs_layout_passes=False),
        scratch_types=[
            pltpu.VMEM((FWP,), jnp.float32),
            pltpu.VMEM((FWP,), jnp.float32),
            pltpu.VMEM((FWP,), jnp.float32),
            pltpu.VMEM((NBLK_P,), jnp.float32),
            pltpu.VMEM((64,), jnp.float32),
            pltpu.VMEM((256,), jnp.float32),
            pltpu.VMEM((128,), jnp.float32),
        ],
    )(m1, p01, p11)

    nms_positions = pos_flat.reshape(B, 64, 4)[:, :MAX_OUT, :3]
    nms_scores = sc_flat.reshape(B, 64, 2)[:, :MAX_OUT, :]
    return nms_positions, nms_scores


# speedup vs baseline: 4.7259x; 1.1058x over previous
import functools

import jax
import jax.numpy as jnp
import numpy as np
from jax import lax
from jax.experimental import pallas as pl
from jax.experimental.pallas import tpu as pltpu
from jax.experimental.pallas import tpu_sc as plsc

B = 8
FW = 20000
FWP = 20096
NBLK = 157
NBLK_P = 160
BLK = 128
MAX_OUT = 50
NEG = np.float32(-1e30)
NEG_HALF = np.float32(-5e29)
DIST = np.float32(16.0)
THR = np.float32(0.7)
LOG2E = np.float32(1.4426950408889634)
SEL_PAD = np.float32(3e38)


def _prep_body(logit_ref, d0_ref, d1_ref, wl_ref, m_ref, p0_ref, p1_ref,
               bm_ref):
    x = logit_ref[...]
    e = jnp.exp2(x * (-LOG2E))
    s = 1.0 / (1.0 + e)
    iot = lax.broadcasted_iota(jnp.int32, (B, FW), 1).astype(jnp.float32)
    ic = (iot + 0.5) * 16.0
    p0 = d0_ref[...] * 16.0 + ic
    p1 = d1_ref[...] * 16.0 + ic
    wl = wl_ref[0]
    valid = (p0 >= 0.0) & (p0 <= wl) & (p1 >= 0.0) & (p1 <= wl) & (s >= THR)
    mm = jnp.where(valid, s, NEG)
    m_ref[:, :FW] = mm
    m_ref[:, FW:] = jnp.full((B, FWP - FW), NEG, jnp.float32)
    p0_ref[:, :FW] = p0
    p0_ref[:, FW:] = jnp.zeros((B, FWP - FW), jnp.float32)
    p1_ref[:, :FW] = p1
    p1_ref[:, FW:] = jnp.zeros((B, FWP - FW), jnp.float32)
    mp = jnp.concatenate(
        [mm, jnp.full((B, FWP - FW), NEG, jnp.float32)], axis=1)
    bm = jnp.max(mp.reshape(B, NBLK, BLK), axis=2)
    bm_ref[:, :NBLK] = bm
    bm_ref[:, NBLK:] = jnp.full((B, NBLK_P - NBLK), NEG, jnp.float32)


def _st1(ref, idx, val):
    iota = lax.broadcasted_iota(jnp.int32, (16,), 0)
    idxv = jnp.zeros((16,), jnp.int32) + idx
    valv = jnp.zeros((16,), jnp.float32) + val
    plsc.store_scatter(ref, [idxv], valv, mask=iota == 0)


def _nms_walk_body(m_hbm, p0_hbm, p1_hbm, bm_hbm, pos_hbm, sc_hbm,
                   m_loc, p0_loc, p1_loc, bm_loc, selc_loc,
                   pos_loc, sc_loc, sem):
    wid = lax.axis_index("s") * 2 + lax.axis_index("c")

    @pl.when(wid < B)
    def _run():
        b = wid
        row = pl.ds(pl.multiple_of(b * FWP, 8), FWP)
        cp_m = pltpu.make_async_copy(m_hbm.at[row], m_loc, sem)
        cp_p0 = pltpu.make_async_copy(p0_hbm.at[row], p0_loc, sem)
        cp_p1 = pltpu.make_async_copy(p1_hbm.at[row], p1_loc, sem)
        bmrow = pl.ds(pl.multiple_of(b * NBLK_P, 8), NBLK_P)
        cp_bm = pltpu.make_async_copy(bm_hbm.at[bmrow], bm_loc, sem)
        cp_m.start()
        cp_p0.start()
        cp_p1.start()
        cp_bm.start()

        zero16 = jnp.zeros((16,), jnp.float32)
        for i in range(16):
            pos_loc[pl.ds(i * 16, 16)] = zero16
        for i in range(8):
            sc_loc[pl.ds(i * 16, 16)] = zero16
        for i in range(4):
            selc_loc[pl.ds(i * 16, 16)] = jnp.full((16,), SEL_PAD, jnp.float32)

        cp_m.wait()
        cp_p0.wait()
        cp_p1.wait()
        cp_bm.wait()

        iota16 = lax.broadcasted_iota(jnp.int32, (16,), 0)
        big = jnp.int32(10**9)

        def _cond(carry):
            cnt, done = carry
            return (cnt < MAX_OUT) & jnp.logical_not(done)

        def _step(carry):
            cnt, _ = carry
            gmv = [bm_loc[pl.ds(j * 16, 16)] for j in range(10)]
            gm = gmv[0]
            for j in range(1, 10):
                gm = jnp.maximum(gm, gmv[j])
            mval = jnp.max(gm)
            exhausted = mval <= NEG_HALF
            bidx = jnp.full((16,), big, jnp.int32)
            for j in range(10):
                bidx = jnp.minimum(
                    bidx, jnp.where(gmv[j] == mval, j * 16 + iota16, big))
            b_star = jnp.min(bidx)
            base = b_star * BLK
            mv = [m_loc[pl.ds(base + j * 16, 16)] for j in range(8)]
            qv = jnp.full((16,), big, jnp.int32)
            for j in range(8):
                qv = jnp.minimum(
                    qv, jnp.where(mv[j] == mval, base + j * 16 + iota16, big))
            q = jnp.min(qv)
            zeros16 = jnp.zeros((16,), jnp.int32)
            p0v = plsc.load_gather(p0_loc, [q + zeros16])
            p1v = plsc.load_gather(p1_loc, [q + zeros16])
            cvec = (p0v + p1v) * 0.5
            dm = jnp.abs(selc_loc[pl.ds(0, 16)] - cvec)
            for j in range(1, 4):
                dm = jnp.minimum(
                    dm, jnp.abs(selc_loc[pl.ds(j * 16, 16)] - cvec))
            keep = (jnp.min(dm) > DIST) & jnp.logical_not(exhausted)

            @pl.when(jnp.logical_not(exhausted))
            def _pop():
                _st1(m_loc, q, NEG)
                acc = jnp.full((16,), NEG, jnp.float32)
                for j in range(8):
                    acc = jnp.maximum(
                        acc, jnp.where(base + j * 16 + iota16 == q,
                                       NEG, mv[j]))
                _st1(bm_loc, b_star, jnp.max(acc))

            @pl.when(keep)
            def _emit():
                iotav = jnp.zeros((16,), jnp.int32) + cnt
                plsc.store_scatter(selc_loc, [iotav], cvec, mask=iota16 == 0)
                posv = jnp.where(iota16 == 0, p0v,
                                 jnp.where(iota16 == 1, p1v, 1.0))
                plsc.store_scatter(pos_loc, [4 * cnt + iota16], posv,
                                   mask=iota16 < 3)
                scv = jnp.where(iota16 == 0, mval, 1.0)
                plsc.store_scatter(sc_loc, [2 * cnt + iota16], scv,
                                   mask=iota16 < 2)

            cnt_next = cnt + jnp.where(keep, 1, 0).astype(jnp.int32)
            return cnt_next, exhausted

        lax.while_loop(_cond, _step, (jnp.int32(0), jnp.bool_(False)))

        pltpu.sync_copy(pos_loc,
                        pos_hbm.at[pl.ds(pl.multiple_of(b * 256, 8), 256)])
        pltpu.sync_copy(sc_loc,
                        sc_hbm.at[pl.ds(pl.multiple_of(b * 128, 8), 128)])


def kernel(pred_cls_logit, pred_delta, img_width):
    wl = (jnp.float32(img_width) - 1.0).reshape(1)
    d0 = pred_delta[:, :, 0]
    d1 = pred_delta[:, :, 1]

    m, p0a, p1a, bma = pl.pallas_call(
        _prep_body,
        out_shape=[
            jax.ShapeDtypeStruct((B, FWP), jnp.float32),
            jax.ShapeDtypeStruct((B, FWP), jnp.float32),
            jax.ShapeDtypeStruct((B, FWP), jnp.float32),
            jax.ShapeDtypeStruct((B, NBLK_P), jnp.float32),
        ],
        in_specs=[
            pl.BlockSpec(memory_space=pltpu.VMEM),
            pl.BlockSpec(memory_space=pltpu.VMEM),
            pl.BlockSpec(memory_space=pltpu.VMEM),
            pl.BlockSpec(memory_space=pltpu.SMEM),
        ],
        out_specs=[
            pl.BlockSpec(memory_space=pltpu.VMEM),
            pl.BlockSpec(memory_space=pltpu.VMEM),
            pl.BlockSpec(memory_space=pltpu.VMEM),
            pl.BlockSpec(memory_space=pltpu.VMEM),
        ],
    )(pred_cls_logit, d0, d1, wl)

    m1 = m.reshape(B * FWP)
    p01 = p0a.reshape(B * FWP)
    p11 = p1a.reshape(B * FWP)
    bm1 = bma.reshape(B * NBLK_P)

    mesh = plsc.VectorSubcoreMesh(core_axis_name="c", subcore_axis_name="s",
                                  num_cores=2)
    pos_flat, sc_flat = pl.kernel(
        _nms_walk_body,
        out_type=[
            jax.ShapeDtypeStruct((B * 256,), jnp.float32),
            jax.ShapeDtypeStruct((B * 128,), jnp.float32),
        ],
        mesh=mesh,
        compiler_params=pltpu.CompilerParams(needs_layout_passes=False),
        scratch_types=[
            pltpu.VMEM((FWP,), jnp.float32),
            pltpu.VMEM((FWP,), jnp.float32),
            pltpu.VMEM((FWP,), jnp.float32),
            pltpu.VMEM((NBLK_P,), jnp.float32),
            pltpu.VMEM((64,), jnp.float32),
            pltpu.VMEM((256,), jnp.float32),
            pltpu.VMEM((128,), jnp.float32),
            pltpu.SemaphoreType.DMA,
        ],
    )(m1, p01, p11, bm1)

    nms_positions = pos_flat.reshape(B, 64, 4)[:, :MAX_OUT, :3]
    nms_scores = sc_flat.reshape(B, 64, 2)[:, :MAX_OUT, :]
    return nms_positions, nms_scores


# speedup vs baseline: 4.7896x; 1.0135x over previous
import functools

import jax
import jax.numpy as jnp
import numpy as np
from jax import lax
from jax.experimental import pallas as pl
from jax.experimental.pallas import tpu as pltpu
from jax.experimental.pallas import tpu_sc as plsc

B = 8
FW = 20000
FWP = 20096
NBLK = 157
NBLK_P = 160
BLK = 128
MAX_OUT = 50
NEG = np.float32(-1e30)
NEG_HALF = np.float32(-5e29)
DIST = np.float32(16.0)
THR = np.float32(0.7)
LOG2E = np.float32(1.4426950408889634)
SEL_PAD = np.float32(3e38)


ROW = 3 * FWP + NBLK_P


def _prep_body(logit_ref, d0_ref, d1_ref, wl_ref, out_ref):
    x = logit_ref[...]
    e = jnp.exp2(x * (-LOG2E))
    s = 1.0 / (1.0 + e)
    iot = lax.broadcasted_iota(jnp.int32, (B, FW), 1).astype(jnp.float32)
    ic = (iot + 0.5) * 16.0
    p0 = d0_ref[...] * 16.0 + ic
    p1 = d1_ref[...] * 16.0 + ic
    wl = wl_ref[0]
    valid = (p0 >= 0.0) & (p0 <= wl) & (p1 >= 0.0) & (p1 <= wl) & (s >= THR)
    mm = jnp.where(valid, s, NEG)
    negpad = jnp.full((B, FWP - FW), NEG, jnp.float32)
    out_ref[:, :FW] = mm
    out_ref[:, FW:FWP] = negpad
    out_ref[:, FWP:FWP + FW] = p0
    out_ref[:, FWP + FW:2 * FWP] = jnp.zeros((B, FWP - FW), jnp.float32)
    out_ref[:, 2 * FWP:2 * FWP + FW] = p1
    out_ref[:, 2 * FWP + FW:3 * FWP] = jnp.zeros((B, FWP - FW), jnp.float32)
    mp = jnp.concatenate([mm, negpad], axis=1)
    bm = jnp.max(mp.reshape(B, NBLK, BLK), axis=2)
    out_ref[:, 3 * FWP:3 * FWP + NBLK] = bm
    out_ref[:, 3 * FWP + NBLK:] = jnp.full((B, NBLK_P - NBLK), NEG,
                                           jnp.float32)


def _st1(ref, idx, val):
    iota = lax.broadcasted_iota(jnp.int32, (16,), 0)
    idxv = jnp.zeros((16,), jnp.int32) + idx
    valv = jnp.zeros((16,), jnp.float32) + val
    plsc.store_scatter(ref, [idxv], valv, mask=iota == 0)


def _nms_walk_body(prep_hbm, pos_hbm, sc_hbm,
                   m_loc, p0_loc, p1_loc, bm_loc, selc_loc,
                   pos_loc, sc_loc, sem):
    wid = lax.axis_index("s") * 2 + lax.axis_index("c")

    @pl.when(wid < B)
    def _run():
        b = wid
        base0 = pl.multiple_of(b * ROW, 8)
        cp_m = pltpu.make_async_copy(
            prep_hbm.at[pl.ds(base0, FWP)], m_loc, sem)
        cp_p0 = pltpu.make_async_copy(
            prep_hbm.at[pl.ds(pl.multiple_of(b * ROW + FWP, 8), FWP)],
            p0_loc, sem)
        cp_p1 = pltpu.make_async_copy(
            prep_hbm.at[pl.ds(pl.multiple_of(b * ROW + 2 * FWP, 8), FWP)],
            p1_loc, sem)
        cp_bm = pltpu.make_async_copy(
            prep_hbm.at[pl.ds(pl.multiple_of(b * ROW + 3 * FWP, 8), NBLK_P)],
            bm_loc, sem)
        cp_m.start()
        cp_p0.start()
        cp_p1.start()
        cp_bm.start()

        zero16 = jnp.zeros((16,), jnp.float32)
        for i in range(16):
            pos_loc[pl.ds(i * 16, 16)] = zero16
        for i in range(8):
            sc_loc[pl.ds(i * 16, 16)] = zero16
        for i in range(4):
            selc_loc[pl.ds(i * 16, 16)] = jnp.full((16,), SEL_PAD, jnp.float32)

        cp_m.wait()
        cp_p0.wait()
        cp_p1.wait()
        cp_bm.wait()

        iota16 = lax.broadcasted_iota(jnp.int32, (16,), 0)
        big = jnp.int32(10**9)

        def _cond(carry):
            cnt, done = carry
            return (cnt < MAX_OUT) & jnp.logical_not(done)

        def _step(carry):
            cnt, _ = carry
            gmv = [bm_loc[pl.ds(j * 16, 16)] for j in range(10)]
            gm = gmv[0]
            for j in range(1, 10):
                gm = jnp.maximum(gm, gmv[j])
            mval = jnp.max(gm)
            exhausted = mval <= NEG_HALF
            bidx = jnp.full((16,), big, jnp.int32)
            for j in range(10):
                bidx = jnp.minimum(
                    bidx, jnp.where(gmv[j] == mval, j * 16 + iota16, big))
            b_star = jnp.min(bidx)
            base = b_star * BLK
            mv = [m_loc[pl.ds(base + j * 16, 16)] for j in range(8)]
            qv = jnp.full((16,), big, jnp.int32)
            for j in range(8):
                qv = jnp.minimum(
                    qv, jnp.where(mv[j] == mval, base + j * 16 + iota16, big))
            q = jnp.min(qv)
            zeros16 = jnp.zeros((16,), jnp.int32)
            p0v = plsc.load_gather(p0_loc, [q + zeros16])
            p1v = plsc.load_gather(p1_loc, [q + zeros16])
            cvec = (p0v + p1v) * 0.5
            dm = jnp.abs(selc_loc[pl.ds(0, 16)] - cvec)
            for j in range(1, 4):
                dm = jnp.minimum(
                    dm, jnp.abs(selc_loc[pl.ds(j * 16, 16)] - cvec))
            keep = (jnp.min(dm) > DIST) & jnp.logical_not(exhausted)

            @pl.when(jnp.logical_not(exhausted))
            def _pop():
                _st1(m_loc, q, NEG)
                acc = jnp.full((16,), NEG, jnp.float32)
                for j in range(8):
                    acc = jnp.maximum(
                        acc, jnp.where(base + j * 16 + iota16 == q,
                                       NEG, mv[j]))
                _st1(bm_loc, b_star, jnp.max(acc))

            @pl.when(keep)
            def _emit():
                iotav = jnp.zeros((16,), jnp.int32) + cnt
                plsc.store_scatter(selc_loc, [iotav], cvec, mask=iota16 == 0)
                posv = jnp.where(iota16 == 0, p0v,
                                 jnp.where(iota16 == 1, p1v, 1.0))
                plsc.store_scatter(pos_loc, [4 * cnt + iota16], posv,
                                   mask=iota16 < 3)
                scv = jnp.where(iota16 == 0, mval, 1.0)
                plsc.store_scatter(sc_loc, [2 * cnt + iota16], scv,
                                   mask=iota16 < 2)

            cnt_next = cnt + jnp.where(keep, 1, 0).astype(jnp.int32)
            return cnt_next, exhausted

        lax.while_loop(_cond, _step, (jnp.int32(0), jnp.bool_(False)))

        pltpu.sync_copy(pos_loc,
                        pos_hbm.at[pl.ds(pl.multiple_of(b * 256, 8), 256)])
        pltpu.sync_copy(sc_loc,
                        sc_hbm.at[pl.ds(pl.multiple_of(b * 128, 8), 128)])


def kernel(pred_cls_logit, pred_delta, img_width):
    wl = (jnp.float32(img_width) - 1.0).reshape(1)
    d0 = pred_delta[:, :, 0]
    d1 = pred_delta[:, :, 1]

    prep = pl.pallas_call(
        _prep_body,
        out_shape=jax.ShapeDtypeStruct((B, ROW), jnp.float32),
        in_specs=[
            pl.BlockSpec(memory_space=pltpu.VMEM),
            pl.BlockSpec(memory_space=pltpu.VMEM),
            pl.BlockSpec(memory_space=pltpu.VMEM),
            pl.BlockSpec(memory_space=pltpu.SMEM),
        ],
        out_specs=pl.BlockSpec(memory_space=pltpu.VMEM),
    )(pred_cls_logit, d0, d1, wl)

    prep1 = prep.reshape(B * ROW)

    mesh = plsc.VectorSubcoreMesh(core_axis_name="c", subcore_axis_name="s",
                                  num_cores=2)
    pos_flat, sc_flat = pl.kernel(
        _nms_walk_body,
        out_type=[
            jax.ShapeDtypeStruct((B * 256,), jnp.float32),
            jax.ShapeDtypeStruct((B * 128,), jnp.float32),
        ],
        mesh=mesh,
        compiler_params=pltpu.CompilerParams(needs_layout_passes=False,
                                             skip_device_barrier=True),
        scratch_types=[
            pltpu.VMEM((FWP,), jnp.float32),
            pltpu.VMEM((FWP,), jnp.float32),
            pltpu.VMEM((FWP,), jnp.float32),
            pltpu.VMEM((NBLK_P,), jnp.float32),
            pltpu.VMEM((64,), jnp.float32),
            pltpu.VMEM((256,), jnp.float32),
            pltpu.VMEM((128,), jnp.float32),
            pltpu.SemaphoreType.DMA,
        ],
    )(prep1,)

    nms_positions = pos_flat.reshape(B, 64, 4)[:, :MAX_OUT, :3]
    nms_scores = sc_flat.reshape(B, 64, 2)[:, :MAX_OUT, :]
    return nms_positions, nms_scores


# speedup vs baseline: 4.9429x; 1.0320x over previous
import functools

import jax
import jax.numpy as jnp
import numpy as np
from jax import lax
from jax.experimental import pallas as pl
from jax.experimental.pallas import tpu as pltpu
from jax.experimental.pallas import tpu_sc as plsc

B = 8
FW = 20000
FWP = 20096
NBLK = 157
NBLK_P = 160
BLK = 128
MAX_OUT = 50
NEG = np.float32(-1e30)
NEG_HALF = np.float32(-5e29)
DIST = np.float32(16.0)
THR = np.float32(0.7)
LOG2E = np.float32(1.4426950408889634)
SEL_PAD = np.float32(3e38)


ROW = 3 * FWP + NBLK_P


def _prep_body(logit_ref, d0_ref, d1_ref, wl_ref, out_ref):
    x = logit_ref[...]
    e = jnp.exp2(x * (-LOG2E))
    s = 1.0 / (1.0 + e)
    iot = lax.broadcasted_iota(jnp.int32, (B, FW), 1).astype(jnp.float32)
    ic = (iot + 0.5) * 16.0
    p0 = d0_ref[...] * 16.0 + ic
    p1 = d1_ref[...] * 16.0 + ic
    wl = wl_ref[0]
    valid = (p0 >= 0.0) & (p0 <= wl) & (p1 >= 0.0) & (p1 <= wl) & (s >= THR)
    mm = jnp.where(valid, s, NEG)
    negpad = jnp.full((B, FWP - FW), NEG, jnp.float32)
    out_ref[:, :FW] = mm
    out_ref[:, FW:FWP] = negpad
    out_ref[:, FWP:FWP + FW] = p0
    out_ref[:, FWP + FW:2 * FWP] = jnp.zeros((B, FWP - FW), jnp.float32)
    out_ref[:, 2 * FWP:2 * FWP + FW] = p1
    out_ref[:, 2 * FWP + FW:3 * FWP] = jnp.zeros((B, FWP - FW), jnp.float32)
    mp = jnp.concatenate([mm, negpad], axis=1)
    bm = jnp.max(mp.reshape(B, NBLK, BLK), axis=2)
    out_ref[:, 3 * FWP:3 * FWP + NBLK] = bm
    out_ref[:, 3 * FWP + NBLK:] = jnp.full((B, NBLK_P - NBLK), NEG,
                                           jnp.float32)


def _st1(ref, idx, val):
    iota = lax.broadcasted_iota(jnp.int32, (16,), 0)
    idxv = jnp.zeros((16,), jnp.int32) + idx
    valv = jnp.zeros((16,), jnp.float32) + val
    plsc.store_scatter(ref, [idxv], valv, mask=iota == 0)


def _nms_walk_body(prep_hbm, out_hbm,
                   m_loc, p0_loc, p1_loc, bm_loc, selc_loc,
                   out_loc, sem):
    wid = lax.axis_index("s") * 2 + lax.axis_index("c")

    @pl.when(wid < B)
    def _run():
        b = wid
        base0 = pl.multiple_of(b * ROW, 8)
        cp_m = pltpu.make_async_copy(
            prep_hbm.at[pl.ds(base0, FWP)], m_loc, sem)
        cp_p0 = pltpu.make_async_copy(
            prep_hbm.at[pl.ds(pl.multiple_of(b * ROW + FWP, 8), FWP)],
            p0_loc, sem)
        cp_p1 = pltpu.make_async_copy(
            prep_hbm.at[pl.ds(pl.multiple_of(b * ROW + 2 * FWP, 8), FWP)],
            p1_loc, sem)
        cp_bm = pltpu.make_async_copy(
            prep_hbm.at[pl.ds(pl.multiple_of(b * ROW + 3 * FWP, 8), NBLK_P)],
            bm_loc, sem)
        cp_m.start()
        cp_p0.start()
        cp_p1.start()
        cp_bm.start()

        zero16 = jnp.zeros((16,), jnp.float32)
        for i in range(24):
            out_loc[pl.ds(i * 16, 16)] = zero16
        for i in range(4):
            selc_loc[pl.ds(i * 16, 16)] = jnp.full((16,), SEL_PAD, jnp.float32)

        cp_m.wait()
        cp_p0.wait()
        cp_p1.wait()
        cp_bm.wait()

        iota16 = lax.broadcasted_iota(jnp.int32, (16,), 0)
        big = jnp.int32(10**9)

        def _cond(carry):
            cnt, done = carry
            return (cnt < MAX_OUT) & jnp.logical_not(done)

        def _step(carry):
            cnt, _ = carry
            gmv = [bm_loc[pl.ds(j * 16, 16)] for j in range(10)]
            gm = gmv[0]
            for j in range(1, 10):
                gm = jnp.maximum(gm, gmv[j])
            mval = jnp.max(gm)
            exhausted = mval <= NEG_HALF
            bidx = jnp.full((16,), big, jnp.int32)
            for j in range(10):
                bidx = jnp.minimum(
                    bidx, jnp.where(gmv[j] == mval, j * 16 + iota16, big))
            b_star = jnp.min(bidx)
            base = b_star * BLK
            mv = [m_loc[pl.ds(base + j * 16, 16)] for j in range(8)]
            qv = jnp.full((16,), big, jnp.int32)
            for j in range(8):
                qv = jnp.minimum(
                    qv, jnp.where(mv[j] == mval, base + j * 16 + iota16, big))
            q = jnp.min(qv)
            zeros16 = jnp.zeros((16,), jnp.int32)
            p0v = plsc.load_gather(p0_loc, [q + zeros16])
            p1v = plsc.load_gather(p1_loc, [q + zeros16])
            cvec = (p0v + p1v) * 0.5
            near = jnp.abs(selc_loc[pl.ds(0, 16)] - cvec) <= DIST
            for j in range(1, 4):
                near = near | (
                    jnp.abs(selc_loc[pl.ds(j * 16, 16)] - cvec) <= DIST)
            nearcnt = plsc.all_reduce_population_count(near)
            if nearcnt.ndim:
                nearcnt = nearcnt[0]
            keep = (nearcnt == 0) & jnp.logical_not(exhausted)

            @pl.when(jnp.logical_not(exhausted))
            def _pop():
                _st1(m_loc, q, NEG)
                acc = jnp.full((16,), NEG, jnp.float32)
                for j in range(8):
                    acc = jnp.maximum(
                        acc, jnp.where(base + j * 16 + iota16 == q,
                                       NEG, mv[j]))
                _st1(bm_loc, b_star, jnp.max(acc))

            @pl.when(keep)
            def _emit():
                iotav = jnp.zeros((16,), jnp.int32) + cnt
                plsc.store_scatter(selc_loc, [iotav], cvec, mask=iota16 == 0)
                posv = jnp.where(iota16 == 0, p0v,
                                 jnp.where(iota16 == 1, p1v, 1.0))
                plsc.store_scatter(out_loc, [4 * cnt + iota16], posv,
                                   mask=iota16 < 3)
                scv = jnp.where(iota16 == 0, mval, 1.0)
                plsc.store_scatter(out_loc, [256 + 2 * cnt + iota16], scv,
                                   mask=iota16 < 2)

            cnt_next = cnt + jnp.where(keep, 1, 0).astype(jnp.int32)
            return cnt_next, exhausted

        lax.while_loop(_cond, _step, (jnp.int32(0), jnp.bool_(False)))

        pltpu.sync_copy(out_loc,
                        out_hbm.at[pl.ds(pl.multiple_of(b * 384, 8), 384)])


def kernel(pred_cls_logit, pred_delta, img_width):
    wl = (jnp.float32(img_width) - 1.0).reshape(1)
    d0 = pred_delta[:, :, 0]
    d1 = pred_delta[:, :, 1]

    prep = pl.pallas_call(
        _prep_body,
        out_shape=jax.ShapeDtypeStruct((B, ROW), jnp.float32),
        in_specs=[
            pl.BlockSpec(memory_space=pltpu.VMEM),
            pl.BlockSpec(memory_space=pltpu.VMEM),
            pl.BlockSpec(memory_space=pltpu.VMEM),
            pl.BlockSpec(memory_space=pltpu.SMEM),
        ],
        out_specs=pl.BlockSpec(memory_space=pltpu.VMEM),
    )(pred_cls_logit, d0, d1, wl)

    prep1 = prep.reshape(B * ROW)

    mesh = plsc.VectorSubcoreMesh(core_axis_name="c", subcore_axis_name="s",
                                  num_cores=2)
    out_flat = pl.kernel(
        _nms_walk_body,
        out_type=jax.ShapeDtypeStruct((B * 384,), jnp.float32),
        mesh=mesh,
        compiler_params=pltpu.CompilerParams(needs_layout_passes=False,
                                             skip_device_barrier=True),
        scratch_types=[
            pltpu.VMEM((FWP,), jnp.float32),
            pltpu.VMEM((FWP,), jnp.float32),
            pltpu.VMEM((FWP,), jnp.float32),
            pltpu.VMEM((NBLK_P,), jnp.float32),
            pltpu.VMEM((64,), jnp.float32),
            pltpu.VMEM((384,), jnp.float32),
            pltpu.SemaphoreType.DMA,
        ],
    )(prep1)

    buf = out_flat.reshape(B, 384)
    nms_positions = buf[:, :256].reshape(B, 64, 4)[:, :MAX_OUT, :3]
    nms_scores = buf[:, 256:].reshape(B, 64, 2)[:, :MAX_OUT, :]
    return nms_positions, nms_scores


# speedup vs baseline: 5.5024x; 1.1132x over previous
import functools

import jax
import jax.numpy as jnp
import numpy as np
from jax import lax
from jax.experimental import pallas as pl
from jax.experimental.pallas import tpu as pltpu
from jax.experimental.pallas import tpu_sc as plsc

B = 8
FW = 20000
FWP = 20096
NBLK = 157
NBLK_P = 160
BLK = 128
MAX_OUT = 50
NEG = np.float32(-1e30)
NEG_HALF = np.float32(-5e29)
DIST = np.float32(16.0)
THR = np.float32(0.7)
LOG2E = np.float32(1.4426950408889634)
SEL_PAD = np.float32(3e38)


KM = NBLK
KP0 = NBLK
KP1 = NBLK
KBM = 2
KTOT = 3 * NBLK + KBM


def _prep_body(logit_ref, d0_ref, d1_ref, wl_ref, out_ref):
    x = logit_ref[...]
    e = jnp.exp2(x * (-LOG2E))
    s = 1.0 / (1.0 + e)
    iot = lax.broadcasted_iota(jnp.int32, (B, FW), 1).astype(jnp.float32)
    ic = (iot + 0.5) * 16.0
    p0 = d0_ref[...] * 16.0 + ic
    p1 = d1_ref[...] * 16.0 + ic
    wl = wl_ref[0]
    valid = (p0 >= 0.0) & (p0 <= wl) & (p1 >= 0.0) & (p1 <= wl) & (s >= THR)
    mm = jnp.where(valid, s, NEG)
    negpad = jnp.full((B, FWP - FW), NEG, jnp.float32)
    zpad = jnp.zeros((B, FWP - FW), jnp.float32)
    mp = jnp.concatenate([mm, negpad], axis=1)
    p0p = jnp.concatenate([p0, zpad], axis=1)
    p1p = jnp.concatenate([p1, zpad], axis=1)
    bm = jnp.max(mp.reshape(B, NBLK, BLK), axis=2)
    bmp = jnp.concatenate(
        [bm, jnp.full((B, 256 - NBLK), NEG, jnp.float32)], axis=1)
    for k in range(NBLK):
        out_ref[k] = mp[:, k * BLK:(k + 1) * BLK]
    for k in range(NBLK):
        out_ref[KM + k] = p0p[:, k * BLK:(k + 1) * BLK]
    for k in range(NBLK):
        out_ref[KM + KP0 + k] = p1p[:, k * BLK:(k + 1) * BLK]
    out_ref[3 * NBLK] = bmp[:, :BLK]
    out_ref[3 * NBLK + 1] = bmp[:, BLK:]


def _st1(ref, idx, val):
    iota = lax.broadcasted_iota(jnp.int32, (16,), 0)
    idxv = jnp.zeros((16,), jnp.int32) + idx
    valv = jnp.zeros((16,), jnp.float32) + val
    plsc.store_scatter(ref, [idxv], valv, mask=iota == 0)


def _st1_2d(ref, r, c, val):
    iota = lax.broadcasted_iota(jnp.int32, (16,), 0)
    rv = jnp.zeros((16,), jnp.int32) + r
    cv = jnp.zeros((16,), jnp.int32) + c
    valv = jnp.zeros((16,), jnp.float32) + val
    plsc.store_scatter(ref, [rv, cv], valv, mask=iota == 0)


def _nms_walk_body(prep_hbm, out_hbm,
                   m_loc, p0_loc, p1_loc, bm_loc, selc_loc,
                   out_loc, sem):
    wid = lax.axis_index("s") * 2 + lax.axis_index("c")

    @pl.when(wid < B)
    def _run():
        b = wid
        cp_m = pltpu.make_async_copy(
            prep_hbm.at[pl.ds(0, NBLK), b], m_loc, sem)
        cp_p0 = pltpu.make_async_copy(
            prep_hbm.at[pl.ds(KM, NBLK), b], p0_loc, sem)
        cp_p1 = pltpu.make_async_copy(
            prep_hbm.at[pl.ds(2 * NBLK, NBLK), b], p1_loc, sem)
        cp_bm = pltpu.make_async_copy(
            prep_hbm.at[pl.ds(3 * NBLK, KBM), b], bm_loc, sem)
        cp_m.start()
        cp_p0.start()
        cp_p1.start()
        cp_bm.start()

        zero16 = jnp.zeros((16,), jnp.float32)
        for i in range(24):
            out_loc[pl.ds(i * 16, 16)] = zero16
        for i in range(4):
            selc_loc[pl.ds(i * 16, 16)] = jnp.full((16,), SEL_PAD, jnp.float32)

        cp_m.wait()
        cp_p0.wait()
        cp_p1.wait()
        cp_bm.wait()

        iota16 = lax.broadcasted_iota(jnp.int32, (16,), 0)
        big = jnp.int32(10**9)

        def _cond(carry):
            cnt, done = carry
            return (cnt < MAX_OUT) & jnp.logical_not(done)

        def _step(carry):
            cnt, _ = carry
            gmv = [bm_loc[(j * 16) // BLK, pl.ds((j * 16) % BLK, 16)]
                   for j in range(10)]
            gm = gmv[0]
            for j in range(1, 10):
                gm = jnp.maximum(gm, gmv[j])
            mval = jnp.max(gm)
            exhausted = mval <= NEG_HALF
            bidx = jnp.full((16,), big, jnp.int32)
            for j in range(10):
                bidx = jnp.minimum(
                    bidx, jnp.where(gmv[j] == mval, j * 16 + iota16, big))
            b_star = jnp.min(bidx)
            mv = [m_loc[b_star, pl.ds(j * 16, 16)] for j in range(8)]
            qv = jnp.full((16,), big, jnp.int32)
            for j in range(8):
                qv = jnp.minimum(
                    qv, jnp.where(mv[j] == mval, j * 16 + iota16, big))
            qc = jnp.min(qv)
            zeros16 = jnp.zeros((16,), jnp.int32)
            bsv = b_star + zeros16
            qcv = qc + zeros16
            p0v = plsc.load_gather(p0_loc, [bsv, qcv])
            p1v = plsc.load_gather(p1_loc, [bsv, qcv])
            cvec = (p0v + p1v) * 0.5
            near = jnp.abs(selc_loc[pl.ds(0, 16)] - cvec) <= DIST
            for j in range(1, 4):
                near = near | (
                    jnp.abs(selc_loc[pl.ds(j * 16, 16)] - cvec) <= DIST)
            nearcnt = plsc.all_reduce_population_count(near)
            if nearcnt.ndim:
                nearcnt = nearcnt[0]
            keep = (nearcnt == 0) & jnp.logical_not(exhausted)

            @pl.when(jnp.logical_not(exhausted))
            def _pop():
                _st1_2d(m_loc, b_star, qc, NEG)
                acc = jnp.full((16,), NEG, jnp.float32)
                for j in range(8):
                    acc = jnp.maximum(
                        acc, jnp.where(j * 16 + iota16 == qc, NEG, mv[j]))
                _st1_2d(bm_loc, b_star >> 7, b_star & 127, jnp.max(acc))

            @pl.when(keep)
            def _emit():
                iotav = jnp.zeros((16,), jnp.int32) + cnt
                plsc.store_scatter(selc_loc, [iotav], cvec, mask=iota16 == 0)
                posv = jnp.where(iota16 == 0, p0v,
                                 jnp.where(iota16 == 1, p1v, 1.0))
                plsc.store_scatter(out_loc, [4 * cnt + iota16], posv,
                                   mask=iota16 < 3)
                scv = jnp.where(iota16 == 0, mval, 1.0)
                plsc.store_scatter(out_loc, [256 + 2 * cnt + iota16], scv,
                                   mask=iota16 < 2)

            cnt_next = cnt + jnp.where(keep, 1, 0).astype(jnp.int32)
            return cnt_next, exhausted

        lax.while_loop(_cond, _step, (jnp.int32(0), jnp.bool_(False)))

        pltpu.sync_copy(out_loc,
                        out_hbm.at[pl.ds(pl.multiple_of(b * 384, 8), 384)])


def kernel(pred_cls_logit, pred_delta, img_width):
    wl = (jnp.float32(img_width) - 1.0).reshape(1)
    d0 = pred_delta[:, :, 0]
    d1 = pred_delta[:, :, 1]

    prep = pl.pallas_call(
        _prep_body,
        out_shape=jax.ShapeDtypeStruct((KTOT, B, BLK), jnp.float32),
        in_specs=[
            pl.BlockSpec(memory_space=pltpu.VMEM),
            pl.BlockSpec(memory_space=pltpu.VMEM),
            pl.BlockSpec(memory_space=pltpu.VMEM),
            pl.BlockSpec(memory_space=pltpu.SMEM),
        ],
        out_specs=pl.BlockSpec(memory_space=pltpu.VMEM),
    )(pred_cls_logit, d0, d1, wl)

    mesh = plsc.VectorSubcoreMesh(core_axis_name="c", subcore_axis_name="s",
                                  num_cores=2)
    out_flat = pl.kernel(
        _nms_walk_body,
        out_type=jax.ShapeDtypeStruct((B * 384,), jnp.float32),
        mesh=mesh,
        compiler_params=pltpu.CompilerParams(needs_layout_passes=False,
                                             skip_device_barrier=True),
        scratch_types=[
            pltpu.VMEM((NBLK, BLK), jnp.float32),
            pltpu.VMEM((NBLK, BLK), jnp.float32),
            pltpu.VMEM((NBLK, BLK), jnp.float32),
            pltpu.VMEM((KBM, BLK), jnp.float32),
            pltpu.VMEM((64,), jnp.float32),
            pltpu.VMEM((384,), jnp.float32),
            pltpu.SemaphoreType.DMA,
        ],
    )(prep)

    buf = out_flat.reshape(B, 384)
    nms_positions = buf[:, :256].reshape(B, 64, 4)[:, :MAX_OUT, :3]
    nms_scores = buf[:, 256:].reshape(B, 64, 2)[:, :MAX_OUT, :]
    return nms_positions, nms_scores
